# baseline (speedup 1.0000x reference)
import jax
import jax.numpy as jnp
from jax import lax
from jax.experimental import pallas as pl
from jax.experimental.pallas import tpu as pltpu
from jax.experimental.pallas import tpu_sc as plsc

_G = 512
_D = 128
_EPS = 1e-05
_B = 10000
_W = 64

_NW = 32
_CH = 3120
_EXTRA = 160


def _stats_kernel(w_ref, x_ref, idx_ref, tbl_ref, acc_ref, cnt_ref):
    i = pl.program_id(0)
    nb = pl.num_programs(0)
    x = x_ref[...]
    idx = idx_ref[0, 0, :]
    b = x.shape[0]
    xb = x.astype(jnp.bfloat16)
    x2 = jnp.concatenate([xb, xb * xb], axis=1)

    @pl.when(i == 0)
    def _():
        acc_ref[...] = jnp.zeros((_G, 2 * _D), jnp.float32)
        cnt_ref[...] = jnp.zeros((_G, 8), jnp.float32)

    w0 = pl.multiple_of(w_ref[i], 8)
    rel = idx - w0
    w_iota = lax.broadcasted_iota(jnp.int32, (_W, b), 0)
    onehot = (w_iota == rel[None, :]).astype(jnp.bfloat16)
    seg = lax.dot_general(
        onehot, x2, (((1,), (0,)), ((), ())),
        preferred_element_type=jnp.float32)
    cnt = jnp.sum(onehot, axis=1, keepdims=True,
                  dtype=jnp.float32)
    acc_ref[pl.ds(w0, _W), :] += seg
    cnt_ref[pl.ds(w0, _W), 0:1] += cnt

    @pl.when(jnp.max(rel) >= _W)
    def _():
        g_iota = lax.broadcasted_iota(jnp.int32, (_G, b), 0)
        far = (rel[None, :] >= _W) & (g_iota == idx[None, :])
        oh_f = far.astype(jnp.bfloat16)
        acc_ref[...] += lax.dot_general(
            oh_f, x2, (((1,), (0,)), ((), ())),
            preferred_element_type=jnp.float32)
        cnt_ref[:, 0:1] += jnp.sum(oh_f, axis=1, keepdims=True,
                                   dtype=jnp.float32)

    @pl.when(i == nb - 1)
    def _():
        s1 = jnp.sum(acc_ref[:, :_D], axis=1, keepdims=True)
        s2 = jnp.sum(acc_ref[:, _D:], axis=1, keepdims=True)
        c = cnt_ref[:, 0:1]
        norm = jnp.maximum(c, 1.0) * float(_D)
        mean = s1 / norm
        var = jnp.maximum(s2 / norm - mean * mean, 0.0)
        rstd = lax.rsqrt(var + _EPS)
        mu = lax.bitcast_convert_type(
            mean.astype(jnp.bfloat16), jnp.uint16).astype(jnp.uint32)
        ru = lax.bitcast_convert_type(
            rstd.astype(jnp.bfloat16), jnp.uint16).astype(jnp.uint32)
        packed = lax.bitcast_convert_type(
            (ru << 16) | mu, jnp.float32)
        l_iota = lax.broadcasted_iota(jnp.int32, (_G, 8), 1)
        stacked = jnp.where(l_iota == 0, packed, 0.0)
        tbl_ref[...] = jnp.transpose(stacked)


def _sc_gather_body(tbl_hbm, idx_hbm, mr_hbm, tbl_v, idx_v, mout_v):
    c = lax.axis_index("c")
    s = lax.axis_index("s")
    wid = c * 16 + s
    base = wid * _CH
    pltpu.sync_copy(tbl_hbm.at[pl.ds(0, _G)], tbl_v)
    pltpu.sync_copy(idx_hbm.at[pl.ds(base, _CH)], idx_v.at[pl.ds(0, _CH)])

    def gbody(i, carry):
        for u in range(15):
            off = i * 240 + u * 16
            iv = idx_v[pl.ds(off, 16)]
            mout_v[pl.ds(off, 16)] = plsc.load_gather(tbl_v, [iv])
        return carry

    lax.fori_loop(0, _CH // 240, gbody, 0)
    pltpu.sync_copy(mout_v.at[pl.ds(0, _CH)], mr_hbm.at[pl.ds(base, _CH)])

    @pl.when(wid == _NW - 1)
    def _():
        tb = _NW * _CH - _CH
        pltpu.sync_copy(idx_hbm.at[pl.ds(tb + _CH, _EXTRA)],
                        idx_v.at[pl.ds(_CH, _EXTRA)])

        def tbody(i, carry):
            off = _CH + i * 16
            iv = idx_v[pl.ds(off, 16)]
            mout_v[pl.ds(off, 16)] = plsc.load_gather(tbl_v, [iv])
            return carry

        lax.fori_loop(0, _EXTRA // 16, tbody, 0)
        pltpu.sync_copy(mout_v.at[pl.ds(_CH, _EXTRA)],
                        mr_hbm.at[pl.ds(tb + _CH, _EXTRA)])


def _norm_kernel(x_ref, p_ref, w_ref, b_ref, out_ref):
    x = x_ref[...]
    p = p_ref[0]
    t = lax.bitcast_convert_type(jnp.transpose(p), jnp.int32)
    mean_c = lax.bitcast_convert_type(t << 16, jnp.float32)
    rstd_c = lax.bitcast_convert_type(t & jnp.int32(-65536), jnp.float32)
    out_ref[...] = (x - mean_c) * (rstd_c * w_ref[0]) + b_ref[0]


@jax.jit
def kernel(x, node_index, weight, bias):
    n, d = x.shape
    nb = n // _B
    idx3 = node_index.reshape(nb, 1, _B)
    wstart = jnp.minimum(node_index[::_B] & ~7, _G - _W).astype(jnp.int32)

    tbl = pl.pallas_call(
        _stats_kernel,
        grid=(nb,),
        in_specs=[
            pl.BlockSpec(memory_space=pltpu.SMEM),
            pl.BlockSpec((_B, d), lambda i: (i, 0)),
            pl.BlockSpec((1, 1, _B), lambda i: (i, 0, 0)),
        ],
        out_specs=pl.BlockSpec((8, _G), lambda i: (0, 0)),
        out_shape=jax.ShapeDtypeStruct((8, _G), jnp.float32),
        scratch_shapes=[
            pltpu.VMEM((_G, 2 * _D), jnp.float32),
            pltpu.VMEM((_G, 8), jnp.float32),
        ],
        compiler_params=pltpu.CompilerParams(
            dimension_semantics=("arbitrary",)),
    )(wstart, x, idx3)

    tbl_flat = tbl.reshape(-1)

    sc_gather = pl.kernel(
        _sc_gather_body,
        out_type=jax.ShapeDtypeStruct((n,), jnp.float32),
        mesh=plsc.VectorSubcoreMesh(
            core_axis_name="c", subcore_axis_name="s",
            num_cores=2, num_subcores=16),
        compiler_params=pltpu.CompilerParams(needs_layout_passes=False),
        scratch_types=[
            pltpu.VMEM((_G,), jnp.float32),
            pltpu.VMEM((_CH + _EXTRA,), jnp.int32),
            pltpu.VMEM((_CH + _EXTRA,), jnp.float32),
        ],
    )
    mr = sc_gather(tbl_flat, node_index).reshape(nb, 1, _B)

    out = pl.pallas_call(
        _norm_kernel,
        grid=(nb,),
        in_specs=[
            pl.BlockSpec((_B, d), lambda i: (i, 0)),
            pl.BlockSpec((1, 1, _B), lambda i: (i, 0, 0)),
            pl.BlockSpec(memory_space=pltpu.SMEM),
            pl.BlockSpec(memory_space=pltpu.SMEM),
        ],
        out_specs=pl.BlockSpec((_B, d), lambda i: (i, 0)),
        out_shape=jax.ShapeDtypeStruct((n, d), jnp.float32),
        compiler_params=pltpu.CompilerParams(
            dimension_semantics=("parallel",)),
    )(x, mr, weight, bias)
    return out


# speedup vs baseline: 1.0391x; 1.0391x over previous
import jax
import jax.numpy as jnp
from jax import lax
from jax.experimental import pallas as pl
from jax.experimental.pallas import tpu as pltpu
from jax.experimental.pallas import tpu_sc as plsc

_G = 512
_D = 128
_EPS = 1e-05
_B = 10000
_W = 64

_NW = 32
_CH = 3120
_EXTRA = 160


def _stats_kernel(w_ref, x_ref, idx_ref, tbl_ref, acc_ref, cnt_ref):
    i = pl.program_id(0)
    nb = pl.num_programs(0)
    x = x_ref[...]
    idx = idx_ref[0, 0, :]
    b = x.shape[0]
    xb = x.astype(jnp.bfloat16)
    x2 = jnp.concatenate([xb, xb * xb], axis=1)

    @pl.when(i == 0)
    def _():
        acc_ref[...] = jnp.zeros((_G, 2 * _D), jnp.float32)
        cnt_ref[...] = jnp.zeros((_G, 8), jnp.float32)

    w0 = pl.multiple_of(w_ref[i], 8)
    rel = idx - w0
    w_iota = lax.broadcasted_iota(jnp.int32, (_W, b), 0)
    onehot = (w_iota == rel[None, :]).astype(jnp.bfloat16)
    seg = lax.dot_general(
        onehot, x2, (((1,), (0,)), ((), ())),
        preferred_element_type=jnp.float32)
    cnt = jnp.sum(onehot, axis=1, keepdims=True,
                  dtype=jnp.float32)
    acc_ref[pl.ds(w0, _W), :] += seg
    cnt_ref[pl.ds(w0, _W), 0:1] += cnt

    @pl.when(jnp.max(rel) >= _W)
    def _():
        g_iota = lax.broadcasted_iota(jnp.int32, (_G, b), 0)
        far = (rel[None, :] >= _W) & (g_iota == idx[None, :])
        oh_f = far.astype(jnp.bfloat16)
        acc_ref[...] += lax.dot_general(
            oh_f, x2, (((1,), (0,)), ((), ())),
            preferred_element_type=jnp.float32)
        cnt_ref[:, 0:1] += jnp.sum(oh_f, axis=1, keepdims=True,
                                   dtype=jnp.float32)

    @pl.when(i == nb - 1)
    def _():
        s1 = jnp.sum(acc_ref[:, :_D], axis=1, keepdims=True)
        s2 = jnp.sum(acc_ref[:, _D:], axis=1, keepdims=True)
        c = cnt_ref[:, 0:1]
        norm = jnp.maximum(c, 1.0) * float(_D)
        mean = s1 / norm
        var = jnp.maximum(s2 / norm - mean * mean, 0.0)
        rstd = lax.rsqrt(var + _EPS)
        mu = lax.bitcast_convert_type(
            mean.astype(jnp.bfloat16), jnp.uint16).astype(jnp.uint32)
        ru = lax.bitcast_convert_type(
            rstd.astype(jnp.bfloat16), jnp.uint16).astype(jnp.uint32)
        packed = lax.bitcast_convert_type(
            (ru << 16) | mu, jnp.float32)
        l_iota = lax.broadcasted_iota(jnp.int32, (_G, 8), 1)
        stacked = jnp.where(l_iota == 0, packed, 0.0)
        tbl_ref[...] = jnp.transpose(stacked)


def _sc_gather_body(tbl_hbm, idx_hbm, mr_hbm, tbl_v, idx_v, mout_v):
    c = lax.axis_index("c")
    s = lax.axis_index("s")
    wid = c * 16 + s
    base = wid * _CH
    pltpu.sync_copy(tbl_hbm.at[pl.ds(0, _G)], tbl_v)
    pltpu.sync_copy(idx_hbm.at[pl.ds(base, _CH)], idx_v.at[pl.ds(0, _CH)])

    def gbody(i, carry):
        for u in range(15):
            off = i * 240 + u * 16
            iv = idx_v[pl.ds(off, 16)]
            mout_v[pl.ds(off, 16)] = plsc.load_gather(tbl_v, [iv])
        return carry

    lax.fori_loop(0, _CH // 240, gbody, 0)
    pltpu.sync_copy(mout_v.at[pl.ds(0, _CH)], mr_hbm.at[pl.ds(base, _CH)])

    @pl.when(wid == _NW - 1)
    def _():
        tb = _NW * _CH - _CH
        pltpu.sync_copy(idx_hbm.at[pl.ds(tb + _CH, _EXTRA)],
                        idx_v.at[pl.ds(_CH, _EXTRA)])

        def tbody(i, carry):
            off = _CH + i * 16
            iv = idx_v[pl.ds(off, 16)]
            mout_v[pl.ds(off, 16)] = plsc.load_gather(tbl_v, [iv])
            return carry

        lax.fori_loop(0, _EXTRA // 16, tbody, 0)
        pltpu.sync_copy(mout_v.at[pl.ds(_CH, _EXTRA)],
                        mr_hbm.at[pl.ds(tb + _CH, _EXTRA)])


def _norm_kernel(x_ref, p_ref, w_ref, b_ref, out_ref):
    x = x_ref[...]
    p = p_ref[0]
    pi = lax.bitcast_convert_type(p, jnp.int32)
    m_lane = lax.bitcast_convert_type(pi << 16, jnp.float32)
    r_lane = lax.bitcast_convert_type(pi & jnp.int32(-65536), jnp.float32)
    s2 = jnp.concatenate([m_lane, r_lane], axis=0).astype(jnp.bfloat16)
    t = jnp.transpose(s2)
    mean_c = t[:, 0:1].astype(jnp.float32)
    rstd_c = t[:, 1:2].astype(jnp.float32)
    out_ref[...] = (x - mean_c) * (rstd_c * w_ref[0]) + b_ref[0]


@jax.jit
def kernel(x, node_index, weight, bias):
    n, d = x.shape
    nb = n // _B
    idx3 = node_index.reshape(nb, 1, _B)
    wstart = jnp.minimum(node_index[::_B] & ~7, _G - _W).astype(jnp.int32)

    tbl = pl.pallas_call(
        _stats_kernel,
        grid=(nb,),
        in_specs=[
            pl.BlockSpec(memory_space=pltpu.SMEM),
            pl.BlockSpec((_B, d), lambda i: (i, 0)),
            pl.BlockSpec((1, 1, _B), lambda i: (i, 0, 0)),
        ],
        out_specs=pl.BlockSpec((8, _G), lambda i: (0, 0)),
        out_shape=jax.ShapeDtypeStruct((8, _G), jnp.float32),
        scratch_shapes=[
            pltpu.VMEM((_G, 2 * _D), jnp.float32),
            pltpu.VMEM((_G, 8), jnp.float32),
        ],
        compiler_params=pltpu.CompilerParams(
            dimension_semantics=("arbitrary",)),
    )(wstart, x, idx3)

    tbl_flat = tbl.reshape(-1)

    sc_gather = pl.kernel(
        _sc_gather_body,
        out_type=jax.ShapeDtypeStruct((n,), jnp.float32),
        mesh=plsc.VectorSubcoreMesh(
            core_axis_name="c", subcore_axis_name="s",
            num_cores=2, num_subcores=16),
        compiler_params=pltpu.CompilerParams(needs_layout_passes=False),
        scratch_types=[
            pltpu.VMEM((_G,), jnp.float32),
            pltpu.VMEM((_CH + _EXTRA,), jnp.int32),
            pltpu.VMEM((_CH + _EXTRA,), jnp.float32),
        ],
    )
    mr = sc_gather(tbl_flat, node_index).reshape(nb, 1, _B)

    out = pl.pallas_call(
        _norm_kernel,
        grid=(nb,),
        in_specs=[
            pl.BlockSpec((_B, d), lambda i: (i, 0)),
            pl.BlockSpec((1, 1, _B), lambda i: (i, 0, 0)),
            pl.BlockSpec(memory_space=pltpu.SMEM),
            pl.BlockSpec(memory_space=pltpu.SMEM),
        ],
        out_specs=pl.BlockSpec((_B, d), lambda i: (i, 0)),
        out_shape=jax.ShapeDtypeStruct((n, d), jnp.float32),
        compiler_params=pltpu.CompilerParams(
            dimension_semantics=("parallel",)),
    )(x, mr, weight, bias)
    return out
